# baseline (speedup 1.0000x reference)
import functools

import jax
import jax.numpy as jnp
from jax import lax
from jax.experimental import pallas as pl
from jax.experimental.pallas import tpu as pltpu
from jax.experimental.pallas import tpu_sc as plsc

N = 4096
L = 16
NC = 2
NS = 16
NW = NC * NS
ROWS_PER_W = N // NW
RBLK = 4
NBLK = ROWS_PER_W // RBLK
NGRP = N // (4 * L)
NWORDS = N // 4

_mesh = plsc.VectorSubcoreMesh(core_axis_name="c", subcore_axis_name="s")


@functools.partial(
    pl.kernel,
    out_type=jax.ShapeDtypeStruct((N, N), jnp.float32),
    mesh=_mesh,
    scratch_types=[
        pltpu.VMEM((RBLK, N), jnp.float32),
        pltpu.VMEM((RBLK, N), jnp.float32),
        pltpu.VMEM((RBLK, NWORDS), jnp.int32),
        pltpu.VMEM((RBLK, NWORDS), jnp.int32),
        pltpu.VMEM((RBLK, N), jnp.float32),
        pltpu.VMEM((RBLK, N), jnp.float32),
        pltpu.SemaphoreType.DMA,
        pltpu.SemaphoreType.DMA,
        pltpu.SemaphoreType.DMA,
        pltpu.SemaphoreType.DMA,
    ],
    compiler_params=pltpu.CompilerParams(needs_layout_passes=False),
)
def _masked_cumsum_sc(x_hbm, m_hbm, out_hbm, xv0, xv1, mv0, mv1, ov0, ov1,
                      sin0, sin1, sout0, sout1):
    wid = lax.axis_index("s") * NC + lax.axis_index("c")
    row0 = wid * ROWS_PER_W

    def blk_row(b):
        return row0 + jnp.minimum(b, NBLK - 1) * RBLK

    def start_in(b, xv, mv, sem):
        r = blk_row(b)
        pltpu.make_async_copy(x_hbm.at[pl.ds(r, RBLK)], xv, sem).start()
        pltpu.make_async_copy(m_hbm.at[pl.ds(r, RBLK)], mv, sem).start()

    def wait_in(xv, mv, sem):
        pltpu.make_async_copy(x_hbm.at[pl.ds(row0, RBLK)], xv, sem).wait()
        pltpu.make_async_copy(m_hbm.at[pl.ds(row0, RBLK)], mv, sem).wait()

    def start_out(b, ov, sem):
        r = blk_row(b)
        pltpu.make_async_copy(ov, out_hbm.at[pl.ds(r, RBLK)], sem).start()

    def wait_out(ov, sem):
        pltpu.make_async_copy(ov, out_hbm.at[pl.ds(row0, RBLK)], sem).wait()

    def compute_block(xv, mv, ov):
        def grp(g, carries):
            carries = list(carries)
            for rr in range(RBLK):
                mw = mv[rr, pl.ds(g * L, L)]
                for b in range(4):
                    mbits = mw if b == 0 else (mw >> (8 * b))
                    mf = (mbits & 1).astype(jnp.float32)
                    sl = pl.ds((4 * g + b) * L, L)
                    masked = xv[rr, sl] * mf
                    s = jnp.cumsum(masked)
                    ov[rr, sl] = s + carries[rr]
                    carries[rr] = s[L - 1] + carries[rr]
            return tuple(carries)

        lax.fori_loop(0, NGRP, grp, (jnp.float32(0.0),) * RBLK)

    def do_pair(k, carry_unused):
        b0 = 2 * k
        b1 = 2 * k + 1
        wait_in(xv0, mv0, sin0)

        @pl.when(k > 0)
        def _():
            wait_out(ov0, sout0)

        compute_block(xv0, mv0, ov0)
        start_out(b0, ov0, sout0)
        start_in(b0 + 2, xv0, mv0, sin0)
        wait_in(xv1, mv1, sin1)

        @pl.when(k > 0)
        def _():
            wait_out(ov1, sout1)

        compute_block(xv1, mv1, ov1)
        start_out(b1, ov1, sout1)
        start_in(b1 + 2, xv1, mv1, sin1)
        return carry_unused

    start_in(0, xv0, mv0, sin0)
    start_in(1, xv1, mv1, sin1)
    lax.fori_loop(0, NBLK // 2, do_pair, 0)
    wait_out(ov0, sout0)
    wait_out(ov1, sout1)
    wait_in(xv0, mv0, sin0)
    wait_in(xv1, mv1, sin1)


def kernel(x, mask):
    m32 = mask.astype(jnp.int32).reshape(N, NGRP, 4, L)
    mw = (m32[:, :, 0, :]
          | (m32[:, :, 1, :] << 8)
          | (m32[:, :, 2, :] << 16)
          | (m32[:, :, 3, :] << 24))
    return _masked_cumsum_sc(x, mw.reshape(N, NWORDS))


# speedup vs baseline: 1.0378x; 1.0378x over previous
import functools

import jax
import jax.numpy as jnp
from jax import lax
from jax.experimental import pallas as pl
from jax.experimental.pallas import tpu as pltpu
from jax.experimental.pallas import tpu_sc as plsc

N = 4096
L = 16
NC = 2
NS = 16
NW = NC * NS
ROWS_PER_W = N // NW
RBLK = 4
NBLK = ROWS_PER_W // RBLK
NGRP = N // (4 * L)
NWORDS = N // 4

_mesh = plsc.VectorSubcoreMesh(core_axis_name="c", subcore_axis_name="s")


@functools.partial(
    pl.kernel,
    out_type=jax.ShapeDtypeStruct((N, N), jnp.float32),
    mesh=_mesh,
    scratch_types=[
        pltpu.VMEM((RBLK, N), jnp.float32),
        pltpu.VMEM((RBLK, N), jnp.float32),
        pltpu.VMEM((RBLK, NWORDS), jnp.int32),
        pltpu.VMEM((RBLK, NWORDS), jnp.int32),
        pltpu.VMEM((RBLK, N), jnp.float32),
        pltpu.VMEM((RBLK, N), jnp.float32),
        pltpu.SemaphoreType.DMA,
        pltpu.SemaphoreType.DMA,
        pltpu.SemaphoreType.DMA,
        pltpu.SemaphoreType.DMA,
    ],
    compiler_params=pltpu.CompilerParams(needs_layout_passes=False),
)
def _masked_cumsum_sc(x_hbm, m_hbm, out_hbm, xv0, xv1, mv0, mv1, ov0, ov1,
                      sin0, sin1, sout0, sout1):
    wid = lax.axis_index("s") * NC + lax.axis_index("c")
    row0 = wid * ROWS_PER_W

    def blk_row(b):
        return row0 + jnp.minimum(b, NBLK - 1) * RBLK

    def start_in(b, xv, mv, sem):
        r = blk_row(b)
        pltpu.make_async_copy(x_hbm.at[pl.ds(r, RBLK)], xv, sem).start()
        pltpu.make_async_copy(m_hbm.at[pl.ds(r, RBLK)], mv, sem).start()

    def wait_in(xv, mv, sem):
        pltpu.make_async_copy(x_hbm.at[pl.ds(row0, RBLK)], xv, sem).wait()
        pltpu.make_async_copy(m_hbm.at[pl.ds(row0, RBLK)], mv, sem).wait()

    def start_out(b, ov, sem):
        r = blk_row(b)
        pltpu.make_async_copy(ov, out_hbm.at[pl.ds(r, RBLK)], sem).start()

    def wait_out(ov, sem):
        pltpu.make_async_copy(ov, out_hbm.at[pl.ds(row0, RBLK)], sem).wait()

    def compute_block(xv, mv, ov):
        def grp(g, carries):
            carries = list(carries)
            for rr in range(RBLK):
                mw = mv[rr, pl.ds(g * L, L)]
                for b in range(4):
                    bit = jnp.int32(1 << (8 * b))
                    sl = pl.ds((4 * g + b) * L, L)
                    masked = jnp.where((mw & bit) != 0, xv[rr, sl], 0.0)
                    s = jnp.cumsum(masked)
                    ov[rr, sl] = s + carries[rr]
                    carries[rr] = s[L - 1] + carries[rr]
            return tuple(carries)

        lax.fori_loop(0, NGRP, grp, (jnp.float32(0.0),) * RBLK)

    def do_pair(k, carry_unused):
        b0 = 2 * k
        b1 = 2 * k + 1
        wait_in(xv0, mv0, sin0)

        @pl.when(k > 0)
        def _():
            wait_out(ov0, sout0)

        compute_block(xv0, mv0, ov0)
        start_out(b0, ov0, sout0)
        start_in(b0 + 2, xv0, mv0, sin0)
        wait_in(xv1, mv1, sin1)

        @pl.when(k > 0)
        def _():
            wait_out(ov1, sout1)

        compute_block(xv1, mv1, ov1)
        start_out(b1, ov1, sout1)
        start_in(b1 + 2, xv1, mv1, sin1)
        return carry_unused

    start_in(0, xv0, mv0, sin0)
    start_in(1, xv1, mv1, sin1)
    lax.fori_loop(0, NBLK // 2, do_pair, 0)
    wait_out(ov0, sout0)
    wait_out(ov1, sout1)
    wait_in(xv0, mv0, sin0)
    wait_in(xv1, mv1, sin1)


def kernel(x, mask):
    m32 = mask.astype(jnp.int32).reshape(N, NGRP, 4, L)
    mw = (m32[:, :, 0, :]
          | (m32[:, :, 1, :] << 8)
          | (m32[:, :, 2, :] << 16)
          | (m32[:, :, 3, :] << 24))
    return _masked_cumsum_sc(x, mw.reshape(N, NWORDS))
